# Initial kernel scaffold; baseline (speedup 1.0000x reference)
import jax
import jax.numpy as jnp
from jax.experimental import pallas as pl


def kernel(x, edge_index, W1l, W1r, b1, W2l, W2r, b2, W3l, W3r, b3):
    raise NotImplementedError("write your pallas kernel here")


# speedup vs baseline: 5.3238x; 5.3238x over previous
import jax
import jax.numpy as jnp
from jax import lax
from jax.experimental import pallas as pl
from jax.experimental.pallas import tpu as pltpu
from jax.experimental.pallas import tpu_sc as plsc

N_NODES = 10000
N_EDGES = 320000
D = 128

NC = 2
NS = 16
EDGES_PER_CORE = N_EDGES // NC
EDGES_PER_TILE = EDGES_PER_CORE // NS
CHUNK = 128
NFULL = EDGES_PER_TILE // CHUNK
TAIL = EDGES_PER_TILE - NFULL * CHUNK
RPT = 640
NPAD = NS * RPT


def _fill_vmem_rows(ref, nrows, ncols, value):
  v16 = jnp.full((16,), value, jnp.float32)

  def body(i, carry):
    for k in range(ncols // 16):
      ref[i, pl.ds(k * 16, 16)] = v16
    return carry

  lax.fori_loop(0, nrows, body, 0)


def _make_sc_agg(with_cnt):
  mesh = plsc.VectorSubcoreMesh(core_axis_name="c", subcore_axis_name="s")

  out_type = [jax.ShapeDtypeStruct((NC * NPAD, D), jnp.float32)]
  if with_cnt:
    out_type.append(jax.ShapeDtypeStruct((NC * NPAD, D), jnp.float32))

  scratch = [
      pltpu.VMEM((CHUNK,), jnp.int32),
      pltpu.VMEM((CHUNK,), jnp.int32),
      pltpu.VMEM((CHUNK, D), jnp.float32),
      pltpu.VMEM((TAIL,), jnp.int32),
      pltpu.VMEM((TAIL,), jnp.int32),
      pltpu.VMEM_SHARED((NPAD, D), jnp.float32),
      pltpu.SemaphoreType.DMA,
  ]

  def body(h_hbm, src_hbm, dst_hbm, agg_out, *rest):
    if with_cnt:
      (cnt_out, src_v, dst_v, rows_v, src_t, dst_t, acc_sh, sem) = rest
    else:
      (src_v, dst_v, rows_v, src_t, dst_t, acc_sh, sem) = rest

    c = lax.axis_index("c")
    s = lax.axis_index("s")
    r0 = pl.multiple_of(s * RPT, 8)
    out0 = pl.multiple_of(c * NPAD + s * RPT, 8)
    base = c * EDGES_PER_CORE + s * EDGES_PER_TILE

    def zero_acc():
      def zinit(j, carry):
        off = pl.multiple_of(r0 + j * CHUNK, 8)
        pltpu.sync_copy(rows_v, acc_sh.at[pl.ds(off, CHUNK)])
        return carry

      lax.fori_loop(0, RPT // CHUNK, zinit, 0)

    if with_cnt:
      _fill_vmem_rows(rows_v, CHUNK, D, 0.0)
      zero_acc()
      _fill_vmem_rows(rows_v, CHUNK, D, 1.0)
      plsc.subcore_barrier()

      def cstep(j, carry):
        off = base + j * CHUNK
        pltpu.sync_copy(dst_hbm.at[pl.ds(off, CHUNK)], dst_v)
        pltpu.sync_copy(rows_v, acc_sh.at[dst_v], add=True)
        return carry

      lax.fori_loop(0, NFULL, cstep, 0)
      offt = base + NFULL * CHUNK
      pltpu.sync_copy(dst_hbm.at[pl.ds(offt, TAIL)], dst_t)
      pltpu.sync_copy(rows_v.at[pl.ds(0, TAIL)], acc_sh.at[dst_t], add=True)
      plsc.subcore_barrier()
      pltpu.sync_copy(acc_sh.at[pl.ds(r0, RPT)],
                      cnt_out.at[pl.ds(out0, RPT)])
      plsc.subcore_barrier()

    _fill_vmem_rows(rows_v, CHUNK, D, 0.0)
    zero_acc()
    plsc.subcore_barrier()

    def step(j, carry):
      off = base + j * CHUNK
      pltpu.sync_copy(src_hbm.at[pl.ds(off, CHUNK)], src_v)
      pltpu.sync_copy(dst_hbm.at[pl.ds(off, CHUNK)], dst_v)
      pltpu.async_copy(h_hbm.at[src_v], rows_v, sem).wait()
      pltpu.sync_copy(rows_v, acc_sh.at[dst_v], add=True)
      return carry

    lax.fori_loop(0, NFULL, step, 0)

    offt = base + NFULL * CHUNK
    pltpu.sync_copy(src_hbm.at[pl.ds(offt, TAIL)], src_t)
    pltpu.sync_copy(dst_hbm.at[pl.ds(offt, TAIL)], dst_t)
    pltpu.async_copy(h_hbm.at[src_t], rows_v.at[pl.ds(0, TAIL)], sem).wait()
    pltpu.sync_copy(rows_v.at[pl.ds(0, TAIL)], acc_sh.at[dst_t], add=True)

    plsc.subcore_barrier()
    pltpu.sync_copy(acc_sh.at[pl.ds(r0, RPT)], agg_out.at[pl.ds(out0, RPT)])

  return pl.kernel(body, out_type=tuple(out_type), mesh=mesh,
                   scratch_types=scratch)


_sc_agg_cnt = _make_sc_agg(True)
_sc_agg = _make_sc_agg(False)


def _make_tc_combine(relu):
  R = 1000

  def body(agg0, agg1, cnt0, cnt1, h, wl, wr, b, out):
    cnt = cnt0[:, 0:1] + cnt1[:, 0:1]
    inv = 1.0 / jnp.maximum(cnt, 1.0)
    agg = (agg0[...] + agg1[...]) * inv
    acc = jnp.dot(agg, wl[...], preferred_element_type=jnp.float32)
    acc = acc + jnp.dot(h[...], wr[...], preferred_element_type=jnp.float32)
    acc = acc + b[...]
    out[...] = jnp.maximum(acc, 0.0) if relu else acc

  row_spec = pl.BlockSpec((R, D), lambda i: (i, 0))
  w_spec = pl.BlockSpec((D, D), lambda i: (0, 0))
  return pl.pallas_call(
      body,
      grid=(N_NODES // R,),
      in_specs=[row_spec, row_spec, row_spec, row_spec, row_spec,
                w_spec, w_spec, pl.BlockSpec((1, D), lambda i: (0, 0))],
      out_specs=row_spec,
      out_shape=jax.ShapeDtypeStruct((N_NODES, D), jnp.float32),
  )


_tc_combine_relu = _make_tc_combine(True)
_tc_combine = _make_tc_combine(False)


def kernel(x, edge_index, W1l, W1r, b1, W2l, W2r, b2, W3l, W3r, b3):
  src = edge_index[0]
  dst = edge_index[1]

  aggp, cntp = _sc_agg_cnt(x, src, dst)
  cnt0, cnt1 = cntp[:N_NODES], cntp[NPAD:NPAD + N_NODES]

  def layer(h, aggp, Wl, Wr, b, relu):
    f = _tc_combine_relu if relu else _tc_combine
    return f(aggp[:N_NODES], aggp[NPAD:NPAD + N_NODES], cnt0, cnt1, h,
             Wl, Wr, b.reshape(1, D))

  h1 = layer(x, aggp, W1l, W1r, b1, True)
  aggp2, = _sc_agg(h1, src, dst)
  h2 = layer(h1, aggp2, W2l, W2r, b2, True)
  aggp3, = _sc_agg(h2, src, dst)
  h3 = layer(h2, aggp3, W3l, W3r, b3, False)
  return h3


# speedup vs baseline: 9.3132x; 1.7494x over previous
import jax
import jax.numpy as jnp
from jax import lax
from jax.experimental import pallas as pl
from jax.experimental.pallas import tpu as pltpu
from jax.experimental.pallas import tpu_sc as plsc

N_NODES = 10000
N_EDGES = 320000
D = 128

NC = 2
NS = 16
CHUNK = 128
NCHUNKS = N_EDGES // CHUNK
CHUNKS_PER_CORE = NCHUNKS // NC
CPT = CHUNKS_PER_CORE // NS
NLEFT = CHUNKS_PER_CORE - NS * CPT
RPT = 640
NPAD = NS * RPT


def _fill_vmem_rows(ref, nrows, ncols, value):
  v16 = jnp.full((16,), value, jnp.float32)

  def body(i, carry):
    for k in range(ncols // 16):
      ref[i, pl.ds(k * 16, 16)] = v16
    return carry

  lax.fori_loop(0, nrows, body, 0)


def _make_sc_agg(with_cnt):
  mesh = plsc.VectorSubcoreMesh(core_axis_name="c", subcore_axis_name="s")

  out_type = [jax.ShapeDtypeStruct((NC * NPAD, D), jnp.float32)]
  if with_cnt:
    out_type.append(jax.ShapeDtypeStruct((NC * NPAD, D), jnp.float32))

  scratch = [
      pltpu.VMEM((2, CHUNK), jnp.int32),
      pltpu.VMEM((2, CHUNK), jnp.int32),
      pltpu.VMEM((CHUNK, D), jnp.float32),
      pltpu.VMEM((CHUNK, D), jnp.float32),
      pltpu.VMEM_SHARED((NPAD, D), jnp.float32),
      pltpu.SemaphoreType.DMA,
      pltpu.SemaphoreType.DMA,
      pltpu.SemaphoreType.DMA,
      pltpu.SemaphoreType.DMA,
  ]

  def body(h_hbm, ep_hbm, agg_out, *rest):
    if with_cnt:
      (cnt_out, sd0, sd1, r0v, r1v, acc_sh, is0, is1, gs0, gs1) = rest
    else:
      (sd0, sd1, r0v, r1v, acc_sh, is0, is1, gs0, gs1) = rest

    c = lax.axis_index("c")
    s = lax.axis_index("s")
    row0 = pl.multiple_of(s * RPT, 8)
    out0 = pl.multiple_of(c * NPAD + s * RPT, 8)
    cbase = c * CHUNKS_PER_CORE + s * CPT
    clast = cbase + CPT - 1
    xtra = jnp.where(s < NLEFT, 1, 0)
    cx = c * CHUNKS_PER_CORE + NS * CPT + s

    def idx_load(jj, sd, sem):
      pltpu.async_copy(ep_hbm.at[jj], sd, sem)

    def idx_wait(sd, sem):
      pltpu.make_async_copy(ep_hbm.at[0], sd, sem).wait()

    def g_start(sd, rv, sem):
      pltpu.async_copy(h_hbm.at[sd.at[0]], rv, sem)

    def g_wait(sd, rv, sem):
      pltpu.make_async_copy(h_hbm.at[sd.at[0]], rv, sem).wait()

    def zero_acc():
      def zinit(j, carry):
        off = pl.multiple_of(row0 + j * CHUNK, 8)
        pltpu.sync_copy(r0v, acc_sh.at[pl.ds(off, CHUNK)])
        return carry

      lax.fori_loop(0, RPT // CHUNK, zinit, 0)

    if with_cnt:
      _fill_vmem_rows(r0v, CHUNK, D, 0.0)
      zero_acc()
      _fill_vmem_rows(r0v, CHUNK, D, 1.0)
      plsc.subcore_barrier()

      idx_load(cbase, sd0, is0)

      def cpair(p, carry):
        j0 = cbase + 2 * p
        idx_load(j0 + 1, sd1, is1)
        idx_wait(sd0, is0)
        pltpu.sync_copy(r0v, acc_sh.at[sd0.at[1]], add=True)
        idx_load(jnp.minimum(j0 + 2, clast), sd0, is0)
        idx_wait(sd1, is1)
        pltpu.sync_copy(r0v, acc_sh.at[sd1.at[1]], add=True)
        return carry

      lax.fori_loop(0, CPT // 2, cpair, 0)
      idx_wait(sd0, is0)

      def cleft(j, carry):
        idx_load(cx, sd0, is0)
        idx_wait(sd0, is0)
        pltpu.sync_copy(r0v, acc_sh.at[sd0.at[1]], add=True)
        return carry

      lax.fori_loop(0, xtra, cleft, 0)
      plsc.subcore_barrier()
      pltpu.sync_copy(acc_sh.at[pl.ds(row0, RPT)],
                      cnt_out.at[pl.ds(out0, RPT)])
      plsc.subcore_barrier()

    _fill_vmem_rows(r0v, CHUNK, D, 0.0)
    zero_acc()
    plsc.subcore_barrier()

    idx_load(cbase, sd0, is0)
    idx_wait(sd0, is0)
    g_start(sd0, r0v, gs0)
    idx_load(cbase + 1, sd1, is1)
    idx_wait(sd1, is1)
    g_start(sd1, r1v, gs1)

    def pair(p, carry):
      j0 = cbase + 2 * p
      g_wait(sd0, r0v, gs0)
      pltpu.sync_copy(r0v, acc_sh.at[sd0.at[1]], add=True)
      idx_load(jnp.minimum(j0 + 2, clast), sd0, is0)
      idx_wait(sd0, is0)
      g_start(sd0, r0v, gs0)
      g_wait(sd1, r1v, gs1)
      pltpu.sync_copy(r1v, acc_sh.at[sd1.at[1]], add=True)
      idx_load(jnp.minimum(j0 + 3, clast), sd1, is1)
      idx_wait(sd1, is1)
      g_start(sd1, r1v, gs1)
      return carry

    lax.fori_loop(0, CPT // 2, pair, 0)
    g_wait(sd0, r0v, gs0)
    g_wait(sd1, r1v, gs1)

    def aleft(j, carry):
      idx_load(cx, sd0, is0)
      idx_wait(sd0, is0)
      g_start(sd0, r0v, gs0)
      g_wait(sd0, r0v, gs0)
      pltpu.sync_copy(r0v, acc_sh.at[sd0.at[1]], add=True)
      return carry

    lax.fori_loop(0, xtra, aleft, 0)

    plsc.subcore_barrier()
    pltpu.sync_copy(acc_sh.at[pl.ds(row0, RPT)], agg_out.at[pl.ds(out0, RPT)])

  return pl.kernel(body, out_type=tuple(out_type), mesh=mesh,
                   scratch_types=scratch)


_sc_agg_cnt = _make_sc_agg(True)
_sc_agg = _make_sc_agg(False)


def _make_tc_combine(relu):
  R = 1000

  def body(agg0, agg1, cnt0, cnt1, h, wl, wr, b, out):
    cnt = cnt0[:, 0:1] + cnt1[:, 0:1]
    inv = 1.0 / jnp.maximum(cnt, 1.0)
    agg = (agg0[...] + agg1[...]) * inv
    acc = jnp.dot(agg, wl[...], preferred_element_type=jnp.float32)
    acc = acc + jnp.dot(h[...], wr[...], preferred_element_type=jnp.float32)
    acc = acc + b[...]
    out[...] = jnp.maximum(acc, 0.0) if relu else acc

  row_spec = pl.BlockSpec((R, D), lambda i: (i, 0))
  w_spec = pl.BlockSpec((D, D), lambda i: (0, 0))
  return pl.pallas_call(
      body,
      grid=(N_NODES // R,),
      in_specs=[row_spec, row_spec, row_spec, row_spec, row_spec,
                w_spec, w_spec, pl.BlockSpec((1, D), lambda i: (0, 0))],
      out_specs=row_spec,
      out_shape=jax.ShapeDtypeStruct((N_NODES, D), jnp.float32),
  )


_tc_combine_relu = _make_tc_combine(True)
_tc_combine = _make_tc_combine(False)


def kernel(x, edge_index, W1l, W1r, b1, W2l, W2r, b2, W3l, W3r, b3):
  src = edge_index[0]
  dst = edge_index[1]
  epairs = jnp.stack(
      [src.reshape(NCHUNKS, CHUNK), dst.reshape(NCHUNKS, CHUNK)], axis=1)

  aggp, cntp = _sc_agg_cnt(x, epairs)
  cnt0, cnt1 = cntp[:N_NODES], cntp[NPAD:NPAD + N_NODES]

  def layer(h, aggp, Wl, Wr, b, relu):
    f = _tc_combine_relu if relu else _tc_combine
    return f(aggp[:N_NODES], aggp[NPAD:NPAD + N_NODES], cnt0, cnt1, h,
             Wl, Wr, b.reshape(1, D))

  h1 = layer(x, aggp, W1l, W1r, b1, True)
  aggp2, = _sc_agg(h1, epairs)
  h2 = layer(h1, aggp2, W2l, W2r, b2, True)
  aggp3, = _sc_agg(h2, epairs)
  h3 = layer(h2, aggp3, W3l, W3r, b3, False)
  return h3


# speedup vs baseline: 10.2533x; 1.1009x over previous
import jax
import jax.numpy as jnp
from jax import lax
from jax.experimental import pallas as pl
from jax.experimental.pallas import tpu as pltpu
from jax.experimental.pallas import tpu_sc as plsc

N_NODES = 10000
N_EDGES = 320000
D = 128

NC = 2
NS = 16
CHUNK = 128
NCHUNKS = N_EDGES // CHUNK
CHUNKS_PER_CORE = NCHUNKS // NC
CPT = CHUNKS_PER_CORE // NS
NLEFT = CHUNKS_PER_CORE - NS * CPT
RPT = 640
NPAD = NS * RPT


def _fill_vmem_rows(ref, nrows, ncols, value):
  v16 = jnp.full((16,), value, jnp.float32)

  def body(i, carry):
    for k in range(ncols // 16):
      ref[i, pl.ds(k * 16, 16)] = v16
    return carry

  lax.fori_loop(0, nrows, body, 0)


def _make_sc_agg(with_cnt):
  mesh = plsc.VectorSubcoreMesh(core_axis_name="c", subcore_axis_name="s")

  out_type = [jax.ShapeDtypeStruct((NC * NPAD, D), jnp.float32)]
  if with_cnt:
    out_type.append(jax.ShapeDtypeStruct((NC * NPAD, D), jnp.float32))

  scratch = [
      pltpu.VMEM((2, CHUNK), jnp.int32),
      pltpu.VMEM((2, CHUNK), jnp.int32),
      pltpu.VMEM((CHUNK,), jnp.int32),
      pltpu.VMEM((CHUNK,), jnp.int32),
      pltpu.VMEM((CHUNK, D), jnp.float32),
      pltpu.VMEM((CHUNK, D), jnp.float32),
      pltpu.VMEM_SHARED((NPAD, D), jnp.float32),
      pltpu.SemaphoreType.DMA,
      pltpu.SemaphoreType.DMA,
      pltpu.SemaphoreType.DMA,
      pltpu.SemaphoreType.DMA,
  ]

  def body(h_hbm, ep_hbm, agg_out, *rest):
    if with_cnt:
      (cnt_out, sd0, sd1, db0, db1, r0v, r1v, acc_sh,
       is0, is1, gs0, gs1) = rest
    else:
      (sd0, sd1, db0, db1, r0v, r1v, acc_sh, is0, is1, gs0, gs1) = rest

    c = lax.axis_index("c")
    s = lax.axis_index("s")
    row0 = pl.multiple_of(s * RPT, 8)
    out0 = pl.multiple_of(c * NPAD + s * RPT, 8)
    cbase = c * CHUNKS_PER_CORE + s * CPT
    clast = cbase + CPT - 1
    xtra = jnp.where(s < NLEFT, 1, 0)
    cx = c * CHUNKS_PER_CORE + NS * CPT + s

    def idx_load(jj, sd, sem):
      pltpu.async_copy(ep_hbm.at[jj], sd, sem)

    def idx_wait(sd, sem):
      pltpu.make_async_copy(ep_hbm.at[0], sd, sem).wait()

    def g_start(sd, rv, sem):
      pltpu.async_copy(h_hbm.at[sd.at[0]], rv, sem)

    def g_wait(sd, rv, sem):
      pltpu.make_async_copy(h_hbm.at[sd.at[0]], rv, sem).wait()

    def copy_dst(sd, db):
      for k in range(CHUNK // 16):
        db[pl.ds(k * 16, 16)] = sd[1, pl.ds(k * 16, 16)]

    def zero_acc():
      def zinit(j, carry):
        off = pl.multiple_of(row0 + j * CHUNK, 8)
        pltpu.async_copy(r0v, acc_sh.at[pl.ds(off, CHUNK)], gs0)
        return carry

      lax.fori_loop(0, RPT // CHUNK, zinit, 0)

      def zdrain(j, carry):
        pltpu.make_async_copy(r0v, acc_sh.at[pl.ds(row0, CHUNK)], gs0).wait()
        return carry

      lax.fori_loop(0, RPT // CHUNK, zdrain, 0)

    if with_cnt:
      _fill_vmem_rows(r0v, CHUNK, D, 0.0)
      zero_acc()
      _fill_vmem_rows(r0v, CHUNK, D, 1.0)
      plsc.subcore_barrier()

      idx_load(cbase, sd0, is0)

      def cpair(p, carry):
        j0 = cbase + 2 * p
        idx_load(j0 + 1, sd1, is1)
        idx_wait(sd0, is0)
        pltpu.sync_copy(r0v, acc_sh.at[sd0.at[1]], add=True)
        idx_load(jnp.minimum(j0 + 2, clast), sd0, is0)
        idx_wait(sd1, is1)
        pltpu.sync_copy(r0v, acc_sh.at[sd1.at[1]], add=True)
        return carry

      lax.fori_loop(0, CPT // 2, cpair, 0)
      idx_wait(sd0, is0)

      def cleft(j, carry):
        idx_load(cx, sd0, is0)
        idx_wait(sd0, is0)
        pltpu.sync_copy(r0v, acc_sh.at[sd0.at[1]], add=True)
        return carry

      lax.fori_loop(0, xtra, cleft, 0)
      plsc.subcore_barrier()
      pltpu.sync_copy(acc_sh.at[pl.ds(row0, RPT)],
                      cnt_out.at[pl.ds(out0, RPT)])
      plsc.subcore_barrier()

    _fill_vmem_rows(r0v, CHUNK, D, 0.0)
    zero_acc()
    plsc.subcore_barrier()

    idx_load(cbase, sd0, is0)
    idx_wait(sd0, is0)
    g_start(sd0, r0v, gs0)
    idx_load(cbase + 1, sd1, is1)
    idx_wait(sd1, is1)
    g_start(sd1, r1v, gs1)

    def pair(p, carry):
      j0 = cbase + 2 * p
      g_wait(sd0, r0v, gs0)
      copy_dst(sd0, db0)
      idx_load(jnp.minimum(j0 + 2, clast), sd0, is0)
      pltpu.sync_copy(r0v, acc_sh.at[db0], add=True)
      idx_wait(sd0, is0)
      g_start(sd0, r0v, gs0)
      g_wait(sd1, r1v, gs1)
      copy_dst(sd1, db1)
      idx_load(jnp.minimum(j0 + 3, clast), sd1, is1)
      pltpu.sync_copy(r1v, acc_sh.at[db1], add=True)
      idx_wait(sd1, is1)
      g_start(sd1, r1v, gs1)
      return carry

    lax.fori_loop(0, CPT // 2, pair, 0)
    g_wait(sd0, r0v, gs0)
    g_wait(sd1, r1v, gs1)

    def aleft(j, carry):
      idx_load(cx, sd0, is0)
      idx_wait(sd0, is0)
      g_start(sd0, r0v, gs0)
      g_wait(sd0, r0v, gs0)
      pltpu.sync_copy(r0v, acc_sh.at[sd0.at[1]], add=True)
      return carry

    lax.fori_loop(0, xtra, aleft, 0)

    plsc.subcore_barrier()
    pltpu.sync_copy(acc_sh.at[pl.ds(row0, RPT)], agg_out.at[pl.ds(out0, RPT)])

  return pl.kernel(body, out_type=tuple(out_type), mesh=mesh,
                   scratch_types=scratch)


_sc_agg_cnt = _make_sc_agg(True)
_sc_agg = _make_sc_agg(False)


def _make_tc_combine(relu):
  R = 1000

  def body(agg0, agg1, cnt0, cnt1, h, wl, wr, b, out):
    cnt = cnt0[:, 0:1] + cnt1[:, 0:1]
    inv = 1.0 / jnp.maximum(cnt, 1.0)
    agg = (agg0[...] + agg1[...]) * inv
    acc = jnp.dot(agg, wl[...], preferred_element_type=jnp.float32)
    acc = acc + jnp.dot(h[...], wr[...], preferred_element_type=jnp.float32)
    acc = acc + b[...]
    out[...] = jnp.maximum(acc, 0.0) if relu else acc

  row_spec = pl.BlockSpec((R, D), lambda i: (i, 0))
  w_spec = pl.BlockSpec((D, D), lambda i: (0, 0))
  return pl.pallas_call(
      body,
      grid=(N_NODES // R,),
      in_specs=[row_spec, row_spec, row_spec, row_spec, row_spec,
                w_spec, w_spec, pl.BlockSpec((1, D), lambda i: (0, 0))],
      out_specs=row_spec,
      out_shape=jax.ShapeDtypeStruct((N_NODES, D), jnp.float32),
  )


_tc_combine_relu = _make_tc_combine(True)
_tc_combine = _make_tc_combine(False)


def kernel(x, edge_index, W1l, W1r, b1, W2l, W2r, b2, W3l, W3r, b3):
  src = edge_index[0]
  dst = edge_index[1]
  epairs = jnp.stack(
      [src.reshape(NCHUNKS, CHUNK), dst.reshape(NCHUNKS, CHUNK)], axis=1)

  aggp, cntp = _sc_agg_cnt(x, epairs)
  cnt0, cnt1 = cntp[:N_NODES], cntp[NPAD:NPAD + N_NODES]

  def layer(h, aggp, Wl, Wr, b, relu):
    f = _tc_combine_relu if relu else _tc_combine
    return f(aggp[:N_NODES], aggp[NPAD:NPAD + N_NODES], cnt0, cnt1, h,
             Wl, Wr, b.reshape(1, D))

  h1 = layer(x, aggp, W1l, W1r, b1, True)
  aggp2, = _sc_agg(h1, epairs)
  h2 = layer(h1, aggp2, W2l, W2r, b2, True)
  aggp3, = _sc_agg(h2, epairs)
  h3 = layer(h2, aggp3, W3l, W3r, b3, False)
  return h3


# speedup vs baseline: 10.9884x; 1.0717x over previous
import jax
import jax.numpy as jnp
from jax import lax
from jax.experimental import pallas as pl
from jax.experimental.pallas import tpu as pltpu
from jax.experimental.pallas import tpu_sc as plsc

N_NODES = 10000
N_EDGES = 320000
D = 128

NC = 2
NS = 16
CHUNK = 128
NCHUNKS = N_EDGES // CHUNK
CHUNKS_PER_CORE = NCHUNKS // NC
CPT = CHUNKS_PER_CORE // NS
NLEFT = CHUNKS_PER_CORE - NS * CPT
RPT = 632
NPAD = NS * RPT


def _fill_vmem_rows(ref, nrows, ncols, value):
  v16 = jnp.full((16,), value, jnp.float32)

  def body(i, carry):
    for k in range(ncols // 16):
      ref[i, pl.ds(k * 16, 16)] = v16
    return carry

  lax.fori_loop(0, nrows, body, 0)


def _make_sc_agg(with_cnt):
  mesh = plsc.VectorSubcoreMesh(core_axis_name="c", subcore_axis_name="s")

  out_type = [jax.ShapeDtypeStruct((NC * NPAD, D), jnp.float32)]
  if with_cnt:
    out_type.append(jax.ShapeDtypeStruct((NC * NPAD, D), jnp.float32))

  scratch = [
      pltpu.VMEM((2, CHUNK), jnp.int32),
      pltpu.VMEM((2, CHUNK), jnp.int32),
      pltpu.VMEM((2, CHUNK), jnp.int32),
      pltpu.VMEM((CHUNK,), jnp.int32),
      pltpu.VMEM((CHUNK, D), jnp.float32),
      pltpu.VMEM((CHUNK, D), jnp.float32),
      pltpu.VMEM((CHUNK, D), jnp.float32),
      pltpu.VMEM_SHARED((NPAD, D), jnp.float32),
      pltpu.SemaphoreType.DMA,
      pltpu.SemaphoreType.DMA,
      pltpu.SemaphoreType.DMA,
      pltpu.SemaphoreType.DMA,
      pltpu.SemaphoreType.DMA,
      pltpu.SemaphoreType.DMA,
  ]

  def body(h_hbm, ep_hbm, agg_out, *rest):
    if with_cnt:
      (cnt_out, sd0, sd1, sd2, db0, r0v, r1v, r2v, acc_sh,
       is0, is1, is2, gs0, gs1, gs2) = rest
    else:
      (sd0, sd1, sd2, db0, r0v, r1v, r2v, acc_sh,
       is0, is1, is2, gs0, gs1, gs2) = rest

    c = lax.axis_index("c")
    s = lax.axis_index("s")
    row0 = pl.multiple_of(s * RPT, 8)
    out0 = pl.multiple_of(c * NPAD + s * RPT, 8)
    cbase = c * CHUNKS_PER_CORE + s * CPT
    clast = cbase + CPT - 1
    xtra = jnp.where(s < NLEFT, 1, 0)
    cx = c * CHUNKS_PER_CORE + NS * CPT + s

    def idx_load(jj, sd, sem):
      pltpu.async_copy(ep_hbm.at[jj], sd, sem)

    def idx_wait(sd, sem):
      pltpu.make_async_copy(ep_hbm.at[0], sd, sem).wait()

    def g_start(sd, rv, sem):
      pltpu.async_copy(h_hbm.at[sd.at[0]], rv, sem)

    def g_wait(sd, rv, sem):
      pltpu.make_async_copy(h_hbm.at[sd.at[0]], rv, sem).wait()

    def copy_dst(sd, db):
      for k in range(CHUNK // 16):
        db[pl.ds(k * 16, 16)] = sd[1, pl.ds(k * 16, 16)]

    def zero_acc():
      def zinit(j, carry):
        off = pl.multiple_of(row0 + j * CHUNK, 8)
        pltpu.async_copy(r0v, acc_sh.at[pl.ds(off, CHUNK)], gs0)
        return carry

      nfull = RPT // CHUNK
      rem = RPT - nfull * CHUNK
      lax.fori_loop(0, nfull, zinit, 0)
      offr = pl.multiple_of(row0 + nfull * CHUNK, 8)
      pltpu.async_copy(r0v.at[pl.ds(0, rem)], acc_sh.at[pl.ds(offr, rem)],
                       gs0)

      def zdrain(j, carry):
        pltpu.make_async_copy(r0v, acc_sh.at[pl.ds(row0, CHUNK)], gs0).wait()
        return carry

      lax.fori_loop(0, nfull, zdrain, 0)
      pltpu.make_async_copy(r0v.at[pl.ds(0, rem)],
                            acc_sh.at[pl.ds(row0, rem)], gs0).wait()

    if with_cnt:
      _fill_vmem_rows(r0v, CHUNK, D, 0.0)
      zero_acc()
      _fill_vmem_rows(r0v, CHUNK, D, 1.0)
      plsc.subcore_barrier()

      idx_load(cbase, sd0, is0)

      def cpair(p, carry):
        j0 = cbase + 2 * p
        idx_load(j0 + 1, sd1, is1)
        idx_wait(sd0, is0)
        pltpu.sync_copy(r0v, acc_sh.at[sd0.at[1]], add=True)
        idx_load(jnp.minimum(j0 + 2, clast), sd0, is0)
        idx_wait(sd1, is1)
        pltpu.sync_copy(r0v, acc_sh.at[sd1.at[1]], add=True)
        return carry

      lax.fori_loop(0, CPT // 2, cpair, 0)
      idx_wait(sd0, is0)

      def cleft(j, carry):
        idx_load(cx, sd0, is0)
        idx_wait(sd0, is0)
        pltpu.sync_copy(r0v, acc_sh.at[sd0.at[1]], add=True)
        return carry

      lax.fori_loop(0, xtra, cleft, 0)
      plsc.subcore_barrier()
      pltpu.sync_copy(acc_sh.at[pl.ds(row0, RPT)],
                      cnt_out.at[pl.ds(out0, RPT)])
      plsc.subcore_barrier()

    _fill_vmem_rows(r0v, CHUNK, D, 0.0)
    zero_acc()
    plsc.subcore_barrier()

    sds = (sd0, sd1, sd2)
    rvs = (r0v, r1v, r2v)
    iss = (is0, is1, is2)
    gss = (gs0, gs1, gs2)
    for i in range(3):
      idx_load(cbase + i, sds[i], iss[i])
      idx_wait(sds[i], iss[i])
      g_start(sds[i], rvs[i], gss[i])

    def triple(p, carry):
      j = cbase + 3 * p
      for i in range(3):
        g_wait(sds[i], rvs[i], gss[i])
        copy_dst(sds[i], db0)
        idx_load(jnp.minimum(j + i + 3, clast), sds[i], iss[i])
        pltpu.sync_copy(rvs[i], acc_sh.at[db0], add=True)
        idx_wait(sds[i], iss[i])
        g_start(sds[i], rvs[i], gss[i])
      return carry

    lax.fori_loop(0, CPT // 3, triple, 0)
    for i in range(3):
      g_wait(sds[i], rvs[i], gss[i])

    def aleft(j, carry):
      idx_load(cx, sd0, is0)
      idx_wait(sd0, is0)
      g_start(sd0, r0v, gs0)
      g_wait(sd0, r0v, gs0)
      pltpu.sync_copy(r0v, acc_sh.at[sd0.at[1]], add=True)
      return carry

    lax.fori_loop(0, xtra, aleft, 0)

    plsc.subcore_barrier()
    pltpu.sync_copy(acc_sh.at[pl.ds(row0, RPT)], agg_out.at[pl.ds(out0, RPT)])

  return pl.kernel(body, out_type=tuple(out_type), mesh=mesh,
                   scratch_types=scratch)


_sc_agg_cnt = _make_sc_agg(True)
_sc_agg = _make_sc_agg(False)


def _make_tc_combine(relu):
  R = 1000

  def body(agg0, agg1, cnt0, cnt1, h, wl, wr, b, out):
    cnt = cnt0[:, 0:1] + cnt1[:, 0:1]
    inv = 1.0 / jnp.maximum(cnt, 1.0)
    agg = (agg0[...] + agg1[...]) * inv
    acc = jnp.dot(agg, wl[...], preferred_element_type=jnp.float32)
    acc = acc + jnp.dot(h[...], wr[...], preferred_element_type=jnp.float32)
    acc = acc + b[...]
    out[...] = jnp.maximum(acc, 0.0) if relu else acc

  row_spec = pl.BlockSpec((R, D), lambda i: (i, 0))
  w_spec = pl.BlockSpec((D, D), lambda i: (0, 0))
  return pl.pallas_call(
      body,
      grid=(N_NODES // R,),
      in_specs=[row_spec, row_spec, row_spec, row_spec, row_spec,
                w_spec, w_spec, pl.BlockSpec((1, D), lambda i: (0, 0))],
      out_specs=row_spec,
      out_shape=jax.ShapeDtypeStruct((N_NODES, D), jnp.float32),
  )


_tc_combine_relu = _make_tc_combine(True)
_tc_combine = _make_tc_combine(False)


def kernel(x, edge_index, W1l, W1r, b1, W2l, W2r, b2, W3l, W3r, b3):
  src = edge_index[0]
  dst = edge_index[1]
  epairs = jnp.stack(
      [src.reshape(NCHUNKS, CHUNK), dst.reshape(NCHUNKS, CHUNK)], axis=1)

  aggp, cntp = _sc_agg_cnt(x, epairs)
  cnt0, cnt1 = cntp[:N_NODES], cntp[NPAD:NPAD + N_NODES]

  def layer(h, aggp, Wl, Wr, b, relu):
    f = _tc_combine_relu if relu else _tc_combine
    return f(aggp[:N_NODES], aggp[NPAD:NPAD + N_NODES], cnt0, cnt1, h,
             Wl, Wr, b.reshape(1, D))

  h1 = layer(x, aggp, W1l, W1r, b1, True)
  aggp2, = _sc_agg(h1, epairs)
  h2 = layer(h1, aggp2, W2l, W2r, b2, True)
  aggp3, = _sc_agg(h2, epairs)
  h3 = layer(h2, aggp3, W3l, W3r, b3, False)
  return h3
